# Initial kernel scaffold; baseline (speedup 1.0000x reference)
import jax
import jax.numpy as jnp
from jax.experimental import pallas as pl


def kernel(feat, pseudo, edge_index, fc_W0, mu0, inv_sigma0, b0, ppW0, ppb0, fc_W1, mu1, inv_sigma1, b1, ppW1, ppb1, fc_W2, mu2, inv_sigma2, b2, ppW2, ppb2):
    raise NotImplementedError("write your pallas kernel here")


# speedup vs baseline: 1.0001x; 1.0001x over previous
import functools

import jax
import jax.numpy as jnp
from jax import lax
from jax.experimental import pallas as pl

N = 10000
E = 160000
IN = 256
HID = 256
CLS = 40
K = 8
DIM = 16
L = 3

_GW_BLK = 1280


def _gw_body(psT_ref, W_ref, pb_ref, mu_ref, isig_ref, out_ref):
    ps = psT_ref[...]
    P = jnp.tanh(
        lax.dot_general(W_ref[...], ps, (((0,), (0,)), ((), ())),
                        preferred_element_type=jnp.float32)
        + pb_ref[...]
    )
    rows = []
    for l in range(L):
        Pl = P[l * DIM:(l + 1) * DIM, :]
        for k in range(K):
            r = l * K + k
            t = (Pl - mu_ref[r, :][:, None]) * isig_ref[r, :][:, None]
            rows.append(jnp.exp(-0.5 * jnp.sum(t * t, axis=0, keepdims=True)))
    out_ref[...] = jnp.concatenate(rows, axis=0)


def _compute_gw(pseudoT, ppW_all, ppb_all, mu_all, isig_all):
    grid = E // _GW_BLK
    return pl.pallas_call(
        _gw_body,
        grid=(grid,),
        in_specs=[
            pl.BlockSpec((2, _GW_BLK), lambda i: (0, i)),
            pl.BlockSpec((2, L * DIM), lambda i: (0, 0)),
            pl.BlockSpec((L * DIM, 1), lambda i: (0, 0)),
            pl.BlockSpec((L * K, DIM), lambda i: (0, 0)),
            pl.BlockSpec((L * K, DIM), lambda i: (0, 0)),
        ],
        out_specs=pl.BlockSpec((L * K, _GW_BLK), lambda i: (0, i)),
        out_shape=jax.ShapeDtypeStruct((L * K, E), jnp.float32),
    )(pseudoT, ppW_all, ppb_all, mu_all, isig_all)


def kernel(feat, pseudo, edge_index, fc_W0, mu0, inv_sigma0, b0, ppW0, ppb0,
           fc_W1, mu1, inv_sigma1, b1, ppW1, ppb1,
           fc_W2, mu2, inv_sigma2, b2, ppW2, ppb2):
    src = edge_index[0]
    dst = edge_index[1]

    pseudoT = pseudo.T
    ppW_all = jnp.concatenate([ppW0, ppW1, ppW2], axis=1)
    ppb_all = jnp.concatenate([ppb0, ppb1, ppb2]).reshape(L * DIM, 1)
    mu_all = jnp.concatenate([mu0, mu1, mu2], axis=0)
    isig_all = jnp.concatenate([inv_sigma0, inv_sigma1, inv_sigma2], axis=0)

    gw_all = _compute_gw(pseudoT, ppW_all, ppb_all, mu_all, isig_all)

    h = feat
    for l, (fc_W, b, out_dim) in enumerate(
            [(fc_W0, b0, HID), (fc_W1, b1, HID), (fc_W2, b2, CLS)]):
        gw = gw_all[l * K:(l + 1) * K, :].T
        hp = (h @ fc_W).reshape(-1, K, out_dim)
        msg = gw[:, :, None] * hp[src]
        agg = jax.ops.segment_sum(msg, dst, num_segments=N)
        h = jnp.sum(agg, axis=1) + b
    return h


# speedup vs baseline: 8.7840x; 8.7833x over previous
import jax
import jax.numpy as jnp
from jax import lax
from jax.experimental import pallas as pl

N = 10000
E = 160000
IN = 256
HID = 256
CLS = 40
K = 8
DIM = 16
L = 3

_GW_BLK = 1280


def _gw_body(psT_ref, W_ref, pb_ref, mu_ref, isig_ref, out_ref):
    ps = psT_ref[...]
    P = jnp.tanh(
        lax.dot_general(W_ref[...], ps, (((0,), (0,)), ((), ())),
                        preferred_element_type=jnp.float32)
        + pb_ref[...]
    )
    rows = []
    for l in range(L):
        Pl = P[l * DIM:(l + 1) * DIM, :]
        for k in range(K):
            r = l * K + k
            t = (Pl - mu_ref[r, :][:, None]) * isig_ref[r, :][:, None]
            rows.append(jnp.exp(-0.5 * jnp.sum(t * t, axis=0, keepdims=True)))
        rows.append(jnp.zeros((16 - K, Pl.shape[1]), jnp.float32))
    out_ref[...] = jnp.concatenate(rows, axis=0)


def _compute_gw(pseudoT, ppW_all, ppb_all, mu_all, isig_all):
    return pl.pallas_call(
        _gw_body,
        grid=(E // _GW_BLK,),
        in_specs=[
            pl.BlockSpec((2, _GW_BLK), lambda i: (0, i)),
            pl.BlockSpec((2, L * DIM), lambda i: (0, 0)),
            pl.BlockSpec((L * DIM, 1), lambda i: (0, 0)),
            pl.BlockSpec((L * K, DIM), lambda i: (0, 0)),
            pl.BlockSpec((L * K, DIM), lambda i: (0, 0)),
        ],
        out_specs=pl.BlockSpec((L * 16, _GW_BLK), lambda i: (0, i)),
        out_shape=jax.ShapeDtypeStruct((L * 16, E), jnp.float32),
    )(pseudoT, ppW_all, ppb_all, mu_all, isig_all)


def _mm_body(a_ref, w_ref, o_ref):
    o_ref[...] = jnp.dot(a_ref[...], w_ref[...],
                         preferred_element_type=jnp.float32)


def _matmul(a, w):
    n, din = a.shape
    dout = w.shape[1]
    BN = 1000
    BO = min(dout, 1024)
    return pl.pallas_call(
        _mm_body,
        grid=(n // BN, dout // BO),
        in_specs=[
            pl.BlockSpec((BN, din), lambda i, j: (i, 0)),
            pl.BlockSpec((din, BO), lambda i, j: (0, j)),
        ],
        out_specs=pl.BlockSpec((BN, BO), lambda i, j: (i, j)),
        out_shape=jax.ShapeDtypeStruct((n, dout), jnp.float32),
    )(a, w)


def _combine_body(gw_ref, hpe_ref, o_ref, *, out_dim):
    gw = gw_ref[...]
    acc = gw[:, 0:1] * hpe_ref[:, pl.ds(0, out_dim)]
    for k in range(1, K):
        acc = acc + gw[:, k:k + 1] * hpe_ref[:, pl.ds(k * out_dim, out_dim)]
    o_ref[...] = acc


def _combine(gw_l, hpe, out_dim, blk):
    import functools
    return pl.pallas_call(
        functools.partial(_combine_body, out_dim=out_dim),
        grid=(E // blk,),
        in_specs=[
            pl.BlockSpec((blk, 16), lambda i: (i, 0)),
            pl.BlockSpec((blk, K * out_dim), lambda i: (i, 0)),
        ],
        out_specs=pl.BlockSpec((blk, out_dim), lambda i: (i, 0)),
        out_shape=jax.ShapeDtypeStruct((E, out_dim), jnp.float32),
    )(gw_l, hpe)


def kernel(feat, pseudo, edge_index, fc_W0, mu0, inv_sigma0, b0, ppW0, ppb0,
           fc_W1, mu1, inv_sigma1, b1, ppW1, ppb1,
           fc_W2, mu2, inv_sigma2, b2, ppW2, ppb2):
    src = edge_index[0]
    dst = edge_index[1]

    pseudoT = pseudo.T
    ppW_all = jnp.concatenate([ppW0, ppW1, ppW2], axis=1)
    ppb_all = jnp.concatenate([ppb0, ppb1, ppb2]).reshape(L * DIM, 1)
    mu_all = jnp.concatenate([mu0, mu1, mu2], axis=0)
    isig_all = jnp.concatenate([inv_sigma0, inv_sigma1, inv_sigma2], axis=0)

    gw_all = _compute_gw(pseudoT, ppW_all, ppb_all, mu_all, isig_all)
    gw_allT = gw_all.T

    h = feat
    layers = [
        (fc_W0, b0, HID, 640),
        (fc_W1, b1, HID, 640),
        (fc_W2, b2, CLS, 1600),
    ]
    for l, (fc_W, b, out_dim, blk) in enumerate(layers):
        hp = _matmul(h, fc_W)
        hpe = jnp.take(hp, src, axis=0)
        gw_l = gw_allT[:, l * 16:(l + 1) * 16]
        m = _combine(gw_l, hpe, out_dim, blk)
        agg = jax.ops.segment_sum(m, dst, num_segments=N)
        h = agg + b
    return h


# speedup vs baseline: 8.9085x; 1.0142x over previous
import jax
import jax.numpy as jnp
from jax import lax
from jax.experimental import pallas as pl

N = 10000
E = 160000
IN = 256
HID = 256
CLS = 40
K = 8
DIM = 16
L = 3

_GW_BLK = 1280


def _gw_body(psT_ref, W_ref, pb_ref, mu_ref, isig_ref, out_ref):
    ps = psT_ref[...]
    P = jnp.tanh(
        lax.dot_general(W_ref[...], ps, (((0,), (0,)), ((), ())),
                        preferred_element_type=jnp.float32)
        + pb_ref[...]
    )
    rows = []
    for l in range(L):
        Pl = P[l * DIM:(l + 1) * DIM, :]
        for k in range(K):
            r = l * K + k
            t = (Pl - mu_ref[r, :][:, None]) * isig_ref[r, :][:, None]
            rows.append(jnp.exp(-0.5 * jnp.sum(t * t, axis=0, keepdims=True)))
        rows.append(jnp.zeros((16 - K, Pl.shape[1]), jnp.float32))
    out_ref[...] = jnp.concatenate(rows, axis=0)


def _compute_gw(pseudoT, ppW_all, ppb_all, mu_all, isig_all):
    return pl.pallas_call(
        _gw_body,
        grid=(E // _GW_BLK,),
        in_specs=[
            pl.BlockSpec((2, _GW_BLK), lambda i: (0, i)),
            pl.BlockSpec((2, L * DIM), lambda i: (0, 0)),
            pl.BlockSpec((L * DIM, 1), lambda i: (0, 0)),
            pl.BlockSpec((L * K, DIM), lambda i: (0, 0)),
            pl.BlockSpec((L * K, DIM), lambda i: (0, 0)),
        ],
        out_specs=pl.BlockSpec((L * 16, _GW_BLK), lambda i: (0, i)),
        out_shape=jax.ShapeDtypeStruct((L * 16, E), jnp.float32),
    )(pseudoT, ppW_all, ppb_all, mu_all, isig_all)


def _mm_body(a_ref, w_ref, o_ref):
    o_ref[...] = jnp.dot(a_ref[...], w_ref[...],
                         preferred_element_type=jnp.float32)


def _matmul(a, w):
    n, din = a.shape
    dout = w.shape[1]
    BN = 1000
    BO = min(dout, 1024)
    return pl.pallas_call(
        _mm_body,
        grid=(n // BN, dout // BO),
        in_specs=[
            pl.BlockSpec((BN, din), lambda i, j: (i, 0)),
            pl.BlockSpec((din, BO), lambda i, j: (0, j)),
        ],
        out_specs=pl.BlockSpec((BN, BO), lambda i, j: (i, j)),
        out_shape=jax.ShapeDtypeStruct((n, dout), jnp.float32),
    )(a, w)


def _combine_body(gw_ref, hpe_ref, o_ref, *, out_dim):
    gw = gw_ref[...]
    acc = gw[:, 0:1] * hpe_ref[:, pl.ds(0, out_dim)]
    for k in range(1, K):
        acc = acc + gw[:, k:k + 1] * hpe_ref[:, pl.ds(k * out_dim, out_dim)]
    o_ref[...] = acc


def _combine(gw_l, hpe, out_dim, blk):
    import functools
    return pl.pallas_call(
        functools.partial(_combine_body, out_dim=out_dim),
        grid=(E // blk,),
        in_specs=[
            pl.BlockSpec((blk, 16), lambda i: (i, 0)),
            pl.BlockSpec((blk, K * out_dim), lambda i: (i, 0)),
        ],
        out_specs=pl.BlockSpec((blk, out_dim), lambda i: (i, 0)),
        out_shape=jax.ShapeDtypeStruct((E, out_dim), jnp.float32),
    )(gw_l, hpe)


def kernel(feat, pseudo, edge_index, fc_W0, mu0, inv_sigma0, b0, ppW0, ppb0,
           fc_W1, mu1, inv_sigma1, b1, ppW1, ppb1,
           fc_W2, mu2, inv_sigma2, b2, ppW2, ppb2):
    src = edge_index[0]
    dst = edge_index[1]

    pseudoT = pseudo.T
    ppW_all = jnp.concatenate([ppW0, ppW1, ppW2], axis=1)
    ppb_all = jnp.concatenate([ppb0, ppb1, ppb2]).reshape(L * DIM, 1)
    mu_all = jnp.concatenate([mu0, mu1, mu2], axis=0)
    isig_all = jnp.concatenate([inv_sigma0, inv_sigma1, inv_sigma2], axis=0)

    gw_all = _compute_gw(pseudoT, ppW_all, ppb_all, mu_all, isig_all)
    gw_allT = gw_all.T

    h = feat
    layers = [
        (fc_W0, b0, HID, 1600),
        (fc_W1, b1, HID, 1600),
        (fc_W2, b2, CLS, 4000),
    ]
    for l, (fc_W, b, out_dim, blk) in enumerate(layers):
        hp = _matmul(h, fc_W)
        hpe = jnp.take(hp, src, axis=0)
        gw_l = gw_allT[:, l * 16:(l + 1) * 16]
        m = _combine(gw_l, hpe, out_dim, blk)
        agg = jax.ops.segment_sum(m, dst, num_segments=N)
        h = agg + b
    return h
